# baseline (speedup 1.0000x reference)
import jax
import jax.numpy as jnp
from jax import lax
from jax.experimental import pallas as pl
from jax.experimental.pallas import tpu as pltpu

B = 16
N = 1024
NFEAT = 128
NHID = 128
S = 16
TAU = 1.0


def _take_lex(va, ia, vb, ib):
    take_a = (va > vb) | ((va == vb) & (ia < ib))
    return jnp.where(take_a, va, vb), jnp.where(take_a, ia, ib)


def _argmax_2d(noisy):
    v = noisy
    n = lax.broadcasted_iota(jnp.int32, (B, N), 1)
    half = N
    while half > 128:
        half //= 2
        v, n = _take_lex(v[:, :half], n[:, :half], v[:, half:], n[:, half:])
    vt = v.T
    nt = n.T
    half = 128
    while half > 1:
        half //= 2
        vt, nt = _take_lex(vt[:half], nt[:half], vt[half:], nt[half:])
    return nt[0]


def _partition_kernel(x_ref, gumbel_ref, mask_ref, w1a_ref, w1b_ref, b1_ref,
                      w2_ref, ctx_w_ref, ctx_b_ref, wih_t_ref, whh_t_ref,
                      bih_ref, bhh_ref, emb_out_ref, sel_out_ref, xw_ref):
    f32 = jnp.float32

    xw_ref[...] = jnp.dot(x_ref[...].reshape(B * N, NFEAT), w1a_ref[...],
                          preferred_element_type=f32).reshape(B, N, NHID)

    xm = jnp.mean(x_ref[...], axis=1)
    gctx = jnp.dot(xm, ctx_w_ref[...], preferred_element_type=f32) \
        + ctx_b_ref[...]

    avail = mask_ref[...] > 0.5
    hidden = jnp.zeros((B, NHID), f32)
    iota_n = lax.broadcasted_iota(jnp.int32, (B, N), 1)
    w2 = w2_ref[...]
    b1 = b1_ref[...]
    bih = bih_ref[...]
    bhh = bhh_ref[...]
    wih_t = wih_t_ref[...]
    whh_t = whh_t_ref[...]

    def gru_step(gi, h):
        gh = jnp.dot(h, whh_t, preferred_element_type=f32) + bhh
        rz = jax.nn.sigmoid(gi[:, :2 * NHID] + gh[:, :2 * NHID])
        r = rz[:, :NHID]
        z = rz[:, NHID:]
        n = jnp.tanh(gi[:, 2 * NHID:] + r * gh[:, 2 * NHID:])
        return (1.0 - z) * n + z * h

    gis = []
    for c in range(S):
        v = jnp.dot(gctx, w1b_ref[...], preferred_element_type=f32) + b1
        h1 = jnp.maximum(xw_ref[...] + v[:, None, :], 0.0)
        scores = jnp.dot(h1.reshape(B * N, NHID), w2,
                         preferred_element_type=f32)
        logits = scores.reshape(B, N)
        logits = jnp.where(avail, logits, -1e9)
        noisy = (logits + gumbel_ref[c]) / TAU

        hp = hidden
        for t in range(c):
            hp = gru_step(gis[t], hp)

        idx = _argmax_2d(noisy)

        rows = [x_ref[b, pl.ds(idx[b], 1), :] for b in range(B)]
        emb = jnp.concatenate(rows, axis=0)
        gis.append(jnp.dot(emb, wih_t, preferred_element_type=f32) + bih)
        hidden = gru_step(gis[c], hp)
        gctx = hidden

        sel = iota_n == idx[:, None]
        sel_f = sel.astype(f32)
        sel_out_ref[c] = sel_f
        emb_out_ref[c] = emb
        avail = jnp.logical_and(avail, jnp.logical_not(sel))


def kernel(x, adj, mask, sel_W1, sel_b1, sel_W2, sel_b2, ctx_W, ctx_b,
           gru_Wih, gru_Whh, gru_bih, gru_bhh):
    del adj, sel_b2
    f32 = jnp.float32
    u = jax.random.uniform(jax.random.key(42), (S, B, N), f32)
    gumbel = -jnp.log(-jnp.log(u + 1e-8) + 1e-8)

    emb_out, sel_out = pl.pallas_call(
        _partition_kernel,
        out_shape=[
            jax.ShapeDtypeStruct((S, B, NFEAT), f32),
            jax.ShapeDtypeStruct((S, B, N), f32),
        ],
        scratch_shapes=[pltpu.VMEM((B, N, NHID), f32)],
    )(
        x,
        gumbel,
        mask.astype(f32),
        sel_W1[:NFEAT],
        sel_W1[NFEAT:],
        sel_b1.reshape(1, NHID),
        sel_W2,
        ctx_W,
        ctx_b.reshape(1, NHID),
        gru_Wih.T,
        gru_Whh.T,
        gru_bih.reshape(1, 3 * NHID),
        gru_bhh.reshape(1, 3 * NHID),
    )

    cf = jnp.transpose(emb_out, (1, 0, 2))
    assign = jnp.transpose(sel_out, (1, 2, 0))
    cadj = jnp.ones((B, S, S), f32) - jnp.eye(S, dtype=f32)[None]
    return cf, cadj, assign
